# baseline (speedup 1.0000x reference)
import functools

import jax
import jax.numpy as jnp
import numpy as np
from jax import lax
from jax.experimental import pallas as pl
from jax.experimental.pallas import tpu as pltpu

_BN_EPS = 1e-5


def _fused_kernel(x_ref, r_ref, w_ref, g_ref, b_ref, o_ref,
                  y_scr, sum_scr, ssq_scr, *, wp, wo, ho, imgs, n):
    p = pl.program_id(0)
    i = pl.program_id(1)
    lout = ho * wp
    ldense = ho * wo
    cout = w_ref.shape[1]

    @pl.when(p == 0)
    def _phase0():
        eye = jnp.eye(cout, dtype=jnp.bfloat16)
        part_sum = jnp.zeros((1, cout), jnp.float32)
        part_ssq = jnp.zeros((1, cout), jnp.float32)
        for im in range(imgs):
            xb = x_ref[im].astype(jnp.bfloat16)
            xrb = lax.dot_general(xb, r_ref[...], (((0,), (0,)), ((), ())),
                                  preferred_element_type=jnp.float32)
            acc = jnp.zeros((cout, lout), jnp.float32)
            for t in range(9):
                s = (t // 3) * wp + (t % 3)
                acc = acc + jnp.dot(w_ref[t], xrb[:, s:s + lout],
                                    preferred_element_type=jnp.float32)
            accb = acc.astype(jnp.bfloat16)
            acct = lax.dot_general(accb, eye, (((0,), (0,)), ((), ())),
                                   preferred_element_type=jnp.float32)
            accc = jnp.concatenate([acct[k * wp:k * wp + wo, :]
                                    for k in range(ho)], axis=0)
            y_scr[pl.ds(i * imgs + im, 1)] = accc.astype(
                jnp.bfloat16).reshape(1, ldense, cout)
            part_sum = part_sum + jnp.sum(accc, axis=0, keepdims=True)
            part_ssq = part_ssq + jnp.sum(accc * accc, axis=0, keepdims=True)

        @pl.when(i == 0)
        def _init():
            sum_scr[...] = part_sum
            ssq_scr[...] = part_ssq

        @pl.when(i > 0)
        def _accum():
            sum_scr[...] = sum_scr[...] + part_sum
            ssq_scr[...] = ssq_scr[...] + part_ssq

    @pl.when(p == 1)
    def _phase1():
        cnt = float(n * ho * wo)
        mean = sum_scr[...] / cnt
        var = jnp.maximum(ssq_scr[...] / cnt - mean * mean, 0.0)
        inv = lax.rsqrt(var + _BN_EPS)
        scale = g_ref[...] * inv
        shift = b_ref[...] - mean * scale
        for im in range(imgs):
            yv = y_scr[i * imgs + im].astype(jnp.float32)
            o_ref[im] = jnp.maximum(yv * scale + shift, 0.0)


def _interp_matrix_np(in_size, out_size):
    if out_size == 1:
        src = np.zeros((1,), np.float32)
    else:
        src = (np.arange(out_size, dtype=np.float32)
               * np.float32((in_size - 1) / (out_size - 1)))
    i0 = np.clip(np.floor(src).astype(np.int32), 0, in_size - 1)
    i1 = np.clip(i0 + 1, 0, in_size - 1)
    frac = (src - i0.astype(np.float32)).astype(np.float32)
    rows = np.arange(out_size)
    m = np.zeros((out_size, in_size), np.float32)
    np.add.at(m, (rows, i0), np.float32(1.0) - frac)
    np.add.at(m, (rows, i1), frac)
    return m


@functools.lru_cache(maxsize=None)
def _resize_const(h, w, ho, wo):
    wp = wo + 2
    hpp = ho + 3
    rh = np.zeros((hpp, h), np.float32)
    rh[1:ho + 1] = _interp_matrix_np(h, ho)
    rw = np.zeros((w, wp), np.float32)
    rw[:, 1:wo + 1] = _interp_matrix_np(w, wo).T
    r_up = np.einsum('ih,wj->hwij', rh, rw).reshape(h * w, hpp * wp)
    return jnp.asarray(r_up.astype(jnp.bfloat16))


@functools.partial(jax.jit, static_argnames=("out_hw",))
def _up_conv_impl(x, conv_w, bn_gamma, bn_beta, *, out_hw):
    n, cin, h, w = x.shape
    ho, wo = out_hw
    cout = conv_w.shape[0]
    hw = h * w
    wp = wo + 2
    hpp = ho + 3
    lpad = hpp * wp
    lout = ho * wp
    ldense = ho * wo

    r_upt = _resize_const(h, w, ho, wo)

    x_t = x.transpose(0, 2, 3, 1).reshape(n, hw, cin)
    w9 = conv_w.transpose(2, 3, 0, 1).reshape(9, cout, cin)
    gamma = bn_gamma.astype(jnp.float32).reshape(1, cout)
    beta = bn_beta.astype(jnp.float32).reshape(1, cout)

    imgs = 4 if n % 4 == 0 else (2 if n % 2 == 0 else 1)
    steps = n // imgs
    vmem_limit = 100 * 1024 * 1024

    kern = functools.partial(_fused_kernel, wp=wp, wo=wo, ho=ho, imgs=imgs,
                             n=n)
    out_nhwc = pl.pallas_call(
        kern,
        grid=(2, steps),
        in_specs=[
            pl.BlockSpec((imgs, hw, cin), lambda p, i: ((1 - p) * i, 0, 0)),
            pl.BlockSpec((hw, lpad), lambda p, i: (0, 0)),
            pl.BlockSpec((9, cout, cin), lambda p, i: (0, 0, 0)),
            pl.BlockSpec((1, cout), lambda p, i: (0, 0)),
            pl.BlockSpec((1, cout), lambda p, i: (0, 0)),
        ],
        out_specs=pl.BlockSpec((imgs, ldense, cout), lambda p, i: (p * i, 0, 0)),
        out_shape=jax.ShapeDtypeStruct((n, ldense, cout), jnp.float32),
        scratch_shapes=[
            pltpu.VMEM((n, ldense, cout), jnp.bfloat16),
            pltpu.VMEM((1, cout), jnp.float32),
            pltpu.VMEM((1, cout), jnp.float32),
        ],
        compiler_params=pltpu.CompilerParams(
            dimension_semantics=("arbitrary", "arbitrary"),
            vmem_limit_bytes=vmem_limit),
        cost_estimate=pl.CostEstimate(
            flops=n * (2 * cin * hw * lpad + 2 * 9 * cout * cin * lout
                       + 2 * cout * cout * lout + 7 * cout * ldense),
            transcendentals=0,
            bytes_accessed=4 * n * cin * hw + 4 * n * cout * ldense
                           + 2 * hw * lpad + 2 * 9 * cout * cin),
    )(x_t, r_upt, w9, gamma, beta)

    return out_nhwc.reshape(n, ho, wo, cout).transpose(0, 3, 1, 2)


def kernel(x, y, conv_w, conv_b, bn_gamma, bn_beta):
    del conv_b
    return _up_conv_impl(x, conv_w, bn_gamma, bn_beta,
                         out_hw=(int(y.shape[2]), int(y.shape[3])))


# speedup vs baseline: 1.3390x; 1.3390x over previous
import functools

import jax
import jax.numpy as jnp
import numpy as np
from jax import lax
from jax.experimental import pallas as pl
from jax.experimental.pallas import tpu as pltpu

_BN_EPS = 1e-5


def _fused_kernel(x_ref, r_ref, w_ref, g_ref, b_ref, o_ref,
                  y_scr, sum_scr, ssq_scr, *, wp, wo, ho, imgs, n):
    p = pl.program_id(0)
    i = pl.program_id(1)
    lout = ho * wp
    ldense = ho * wo
    cout = w_ref.shape[0]

    @pl.when(p == 0)
    def _phase0():
        eye = jnp.eye(cout, dtype=jnp.bfloat16)
        part_sum = jnp.zeros((1, cout), jnp.float32)
        part_ssq = jnp.zeros((1, cout), jnp.float32)
        for im in range(imgs):
            xb = x_ref[im].astype(jnp.bfloat16)
            xr = lax.dot_general(xb, r_ref[...], (((0,), (0,)), ((), ())),
                                 preferred_element_type=jnp.float32)
            xrb = xr.astype(jnp.bfloat16)
            xr9 = jnp.concatenate(
                [xrb[:, (t // 3) * wp + (t % 3):(t // 3) * wp + (t % 3) + lout]
                 for t in range(9)], axis=0)
            acc = lax.dot_general(w_ref[...], xr9, (((1,), (0,)), ((), ())),
                                  preferred_element_type=jnp.float32)
            accb = acc.astype(jnp.bfloat16)
            acct = lax.dot_general(accb, eye, (((0,), (0,)), ((), ())),
                                   preferred_element_type=jnp.float32)
            accc = jnp.concatenate([acct[k * wp:k * wp + wo, :]
                                    for k in range(ho)], axis=0)
            y_scr[pl.ds(i * imgs + im, 1)] = accc.astype(
                jnp.bfloat16).reshape(1, ldense, cout)
            part_sum = part_sum + jnp.sum(accc, axis=0, keepdims=True)
            part_ssq = part_ssq + jnp.sum(accc * accc, axis=0, keepdims=True)

        @pl.when(i == 0)
        def _init():
            sum_scr[...] = part_sum
            ssq_scr[...] = part_ssq

        @pl.when(i > 0)
        def _accum():
            sum_scr[...] = sum_scr[...] + part_sum
            ssq_scr[...] = ssq_scr[...] + part_ssq

    @pl.when(p == 1)
    def _phase1():
        cnt = float(n * ho * wo)
        mean = sum_scr[...] / cnt
        var = jnp.maximum(ssq_scr[...] / cnt - mean * mean, 0.0)
        inv = lax.rsqrt(var + _BN_EPS)
        scale = g_ref[...] * inv
        shift = b_ref[...] - mean * scale
        for im in range(imgs):
            yv = y_scr[i * imgs + im].astype(jnp.float32)
            o_ref[im] = jnp.maximum(yv * scale + shift, 0.0)


def _interp_matrix_np(in_size, out_size):
    if out_size == 1:
        src = np.zeros((1,), np.float32)
    else:
        src = (np.arange(out_size, dtype=np.float32)
               * np.float32((in_size - 1) / (out_size - 1)))
    i0 = np.clip(np.floor(src).astype(np.int32), 0, in_size - 1)
    i1 = np.clip(i0 + 1, 0, in_size - 1)
    frac = (src - i0.astype(np.float32)).astype(np.float32)
    rows = np.arange(out_size)
    m = np.zeros((out_size, in_size), np.float32)
    np.add.at(m, (rows, i0), np.float32(1.0) - frac)
    np.add.at(m, (rows, i1), frac)
    return m


@functools.lru_cache(maxsize=None)
def _resize_const(h, w, ho, wo):
    wp = wo + 2
    hpp = ho + 3
    rh = np.zeros((hpp, h), np.float32)
    rh[1:ho + 1] = _interp_matrix_np(h, ho)
    rw = np.zeros((w, wp), np.float32)
    rw[:, 1:wo + 1] = _interp_matrix_np(w, wo).T
    r_up = np.einsum('ih,wj->hwij', rh, rw).reshape(h * w, hpp * wp)
    return jnp.asarray(r_up.astype(jnp.bfloat16))


@functools.partial(jax.jit, static_argnames=("out_hw",))
def _up_conv_impl(x, conv_w, bn_gamma, bn_beta, *, out_hw):
    n, cin, h, w = x.shape
    ho, wo = out_hw
    cout = conv_w.shape[0]
    hw = h * w
    wp = wo + 2
    hpp = ho + 3
    lpad = hpp * wp
    lout = ho * wp
    ldense = ho * wo

    r_upt = _resize_const(h, w, ho, wo)

    x_t = x.transpose(0, 2, 3, 1).reshape(n, hw, cin)
    w9 = conv_w.astype(jnp.bfloat16).transpose(0, 2, 3, 1).reshape(cout, 9 * cin)
    gamma = bn_gamma.astype(jnp.float32).reshape(1, cout)
    beta = bn_beta.astype(jnp.float32).reshape(1, cout)

    imgs = 4 if n % 4 == 0 else (2 if n % 2 == 0 else 1)
    steps = n // imgs
    vmem_limit = 100 * 1024 * 1024

    kern = functools.partial(_fused_kernel, wp=wp, wo=wo, ho=ho, imgs=imgs,
                             n=n)
    out_nhwc = pl.pallas_call(
        kern,
        grid=(2, steps),
        in_specs=[
            pl.BlockSpec((imgs, hw, cin), lambda p, i: ((1 - p) * i, 0, 0)),
            pl.BlockSpec((hw, lpad), lambda p, i: (0, 0)),
            pl.BlockSpec((cout, 9 * cin), lambda p, i: (0, 0)),
            pl.BlockSpec((1, cout), lambda p, i: (0, 0)),
            pl.BlockSpec((1, cout), lambda p, i: (0, 0)),
        ],
        out_specs=pl.BlockSpec((imgs, ldense, cout), lambda p, i: (p * i, 0, 0)),
        out_shape=jax.ShapeDtypeStruct((n, ldense, cout), jnp.float32),
        scratch_shapes=[
            pltpu.VMEM((n, ldense, cout), jnp.bfloat16),
            pltpu.VMEM((1, cout), jnp.float32),
            pltpu.VMEM((1, cout), jnp.float32),
        ],
        compiler_params=pltpu.CompilerParams(
            dimension_semantics=("arbitrary", "arbitrary"),
            vmem_limit_bytes=vmem_limit),
        cost_estimate=pl.CostEstimate(
            flops=n * (2 * cin * hw * lpad + 2 * 9 * cout * cin * lout
                       + 2 * cout * cout * lout + 7 * cout * ldense),
            transcendentals=0,
            bytes_accessed=4 * n * cin * hw + 4 * n * cout * ldense
                           + 2 * hw * lpad + 2 * 9 * cout * cin),
    )(x_t, r_upt, w9, gamma, beta)

    return out_nhwc.reshape(n, ho, wo, cout).transpose(0, 3, 1, 2)


def kernel(x, y, conv_w, conv_b, bn_gamma, bn_beta):
    del conv_b
    return _up_conv_impl(x, conv_w, bn_gamma, bn_beta,
                         out_hw=(int(y.shape[2]), int(y.shape[3])))
